# Initial kernel scaffold; baseline (speedup 1.0000x reference)
import jax
import jax.numpy as jnp
from jax.experimental import pallas as pl


def kernel(x, W1, b1, ln1_g, ln1_b, W2, b2, W3, b3, Wg1, bg1, Wg2, bg2, Wt, bt, lnt_g, lnt_b):
    raise NotImplementedError("write your pallas kernel here")


# speedup vs baseline: 24.3584x; 24.3584x over previous
import functools

import jax
import jax.numpy as jnp
from jax.experimental import pallas as pl
from jax.experimental.pallas import tpu as pltpu

B = 32
IN_DIM = 8192
HID = 256
K = 4096


def _fused_body(x_ref, W1_ref, b1_ref, ln1_g_ref, ln1_b_ref, W2_ref, b2_ref,
                W3_ref, b3_ref, Wg1_ref, bg1_ref, Wg2_ref, bg2_ref,
                Wt_ref, bt_ref, lnt_g_ref, lnt_b_ref,
                t_out_ref, p_out_ref):
    x = x_ref[...]

    h = jnp.dot(x, W1_ref[...], preferred_element_type=jnp.float32) + b1_ref[...]
    m = jnp.mean(h, axis=-1, keepdims=True)
    v = jnp.mean((h - m) ** 2, axis=-1, keepdims=True)
    h = (h - m) * jax.lax.rsqrt(v + 1e-5) * ln1_g_ref[...] + ln1_b_ref[...]
    h = jnp.maximum(h, 0.0)
    h = jnp.maximum(jnp.dot(h, W2_ref[...], preferred_element_type=jnp.float32) + b2_ref[...], 0.0)
    imp = jnp.dot(h, W3_ref[...], preferred_element_type=jnp.float32) + b3_ref[...]

    g = jnp.maximum(jnp.dot(x, Wg1_ref[...], preferred_element_type=jnp.float32) + bg1_ref[...], 0.0)
    gz = jnp.dot(g, Wg2_ref[...], preferred_element_type=jnp.float32) + bg2_ref[...]
    gates = 1.0 / (1.0 + jnp.exp(-gz))

    sel = imp * gates

    bits = jax.lax.bitcast_convert_type(sel, jnp.int32)
    key = jnp.where(bits >= 0, bits, bits ^ jnp.int32(0x7FFFFFFF))

    def count_ge(c):
        return jnp.sum((key >= c).astype(jnp.int32), axis=1, keepdims=True)

    neg_min = jnp.full((B, 1), jnp.int32(-2147483648))
    zero = jnp.zeros((B, 1), jnp.int32)
    thresh = jnp.where(count_ge(zero) >= K, zero, neg_min)

    def body(i, th):
        bit = jnp.int32(1) << (jnp.int32(30) - i)
        cand = th | bit
        return jnp.where(count_ge(cand) >= K, cand, th)

    thresh = jax.lax.fori_loop(0, 31, body, thresh)

    mask = (key >= thresh).astype(jnp.float32)
    p_out_ref[...] = mask

    t = jnp.dot(x * mask, Wt_ref[...], preferred_element_type=jnp.float32) + bt_ref[...]
    mt = jnp.mean(t, axis=-1, keepdims=True)
    vt = jnp.mean((t - mt) ** 2, axis=-1, keepdims=True)
    t = (t - mt) * jax.lax.rsqrt(vt + 1e-5) * lnt_g_ref[...] + lnt_b_ref[...]
    t_out_ref[...] = jnp.maximum(t, 0.0)


@functools.partial(jax.jit, static_argnames=("interpret",))
def _run(args, interpret=False):
    out_shapes = (
        jax.ShapeDtypeStruct((B, HID), jnp.float32),
        jax.ShapeDtypeStruct((B, IN_DIM), jnp.float32),
    )
    return pl.pallas_call(
        _fused_body,
        out_shape=out_shapes,
        compiler_params=pltpu.CompilerParams(
            vmem_limit_bytes=100 * 1024 * 1024,
        ),
        interpret=interpret,
    )(*args)


def kernel(x, W1, b1, ln1_g, ln1_b, W2, b2, W3, b3, Wg1, bg1, Wg2, bg2, Wt, bt, lnt_g, lnt_b):
    args = (
        x, W1, b1.reshape(1, -1), ln1_g.reshape(1, -1), ln1_b.reshape(1, -1),
        W2, b2.reshape(1, -1), W3, b3.reshape(1, -1),
        Wg1, bg1.reshape(1, -1), Wg2, bg2.reshape(1, -1),
        Wt, bt.reshape(1, -1), lnt_g.reshape(1, -1), lnt_b.reshape(1, -1),
    )
    transformed, selection_probs = _run(args)
    return transformed, selection_probs
